# Initial kernel scaffold; baseline (speedup 1.0000x reference)
import jax
import jax.numpy as jnp
from jax.experimental import pallas as pl


def kernel(raw_cls_bbox, raw_prob):
    raise NotImplementedError("write your pallas kernel here")


# speedup vs baseline: 77.3577x; 77.3577x over previous
import functools

import jax
import jax.numpy as jnp
from jax import lax
from jax.experimental import pallas as pl
from jax.experimental.pallas import tpu as pltpu
from jax.experimental.pallas import tpu_sc as plsc

N = 5000
NCLS = 21
NFG = NCLS - 1
PROB_THRESH = 0.05
NMS_THRESH = 0.3
L = 16
NP = 5024
NCHUNKS = NP // L
CM_CAP = 320
CM_VREGS = CM_CAP // L
NUM_CORES = 2
BIG_I32 = 2**30

_mesh = plsc.VectorSubcoreMesh(core_axis_name="c", subcore_axis_name="s")


def _nms_body(scores_hbm, boxes_hbm, out_hbm,
              s_v, y1_v, x1_v, y2_v, x2_v,
              w_v, cy1_v, cx1_v, cy2_v, cx2_v, cidx_v, cm_v,
              o0_v, o1_v, o2_v, o3_v, o4_v):
    wid = lax.axis_index("s") * NUM_CORES + lax.axis_index("c")

    @pl.when(wid < NFG)
    def _run():
        lane_iota = lax.iota(jnp.int32, L)
        lane0 = lane_iota == 0
        neg1 = jnp.full((L,), -1.0, jnp.float32)
        zero16 = jnp.zeros((L,), jnp.float32)

        pltpu.sync_copy(scores_hbm.at[wid], s_v)
        pltpu.sync_copy(boxes_hbm.at[wid * 4 + 0], y1_v)
        pltpu.sync_copy(boxes_hbm.at[wid * 4 + 1], x1_v)
        pltpu.sync_copy(boxes_hbm.at[wid * 4 + 2], y2_v)
        pltpu.sync_copy(boxes_hbm.at[wid * 4 + 3], x2_v)

        def init_body(i, carry):
            d = pl.ds(pl.multiple_of(i * L, L), L)
            w_v[d] = neg1
            o0_v[d] = zero16
            o1_v[d] = zero16
            o2_v[d] = zero16
            o3_v[d] = zero16
            o4_v[d] = zero16
            return carry
        lax.fori_loop(0, NCHUNKS, init_body, 0)

        def cm_init(i, carry):
            cm_v[pl.ds(pl.multiple_of(i * L, L), L)] = neg1
            return carry
        lax.fori_loop(0, CM_VREGS, cm_init, 0)

        def compact_body(i, base):
            d = pl.ds(pl.multiple_of(i * L, L), L)
            sv = s_v[d]
            msk = sv > PROB_THRESH
            pos = plsc.cumsum(msk.astype(jnp.int32))
            cnt = jnp.max(pos)
            dest = pos + (base - 1)
            plsc.store_scatter(w_v, [dest], sv, mask=msk)
            plsc.store_scatter(cy1_v, [dest], y1_v[d], mask=msk)
            plsc.store_scatter(cx1_v, [dest], x1_v[d], mask=msk)
            plsc.store_scatter(cy2_v, [dest], y2_v[d], mask=msk)
            plsc.store_scatter(cx2_v, [dest], x2_v[d], mask=msk)
            plsc.store_scatter(cidx_v, [dest], lane_iota + i * L, mask=msk)
            return base + cnt
        m_count = lax.fori_loop(0, NCHUNKS, compact_body, 0)
        nch = (m_count + (L - 1)) // L

        def cm_seed(i, carry):
            wv = w_v[pl.ds(pl.multiple_of(i * L, L), L)]
            plsc.store_scatter(cm_v, [jnp.full((L,), i, jnp.int32)],
                               jnp.full((L,), jnp.max(wv), jnp.float32),
                               mask=lane0)
            return carry
        lax.fori_loop(0, nch, cm_seed, 0)

        def global_max():
            def b(i, acc):
                return jnp.maximum(acc, cm_v[pl.ds(pl.multiple_of(i * L, L), L)])
            return jnp.max(lax.fori_loop(0, CM_VREGS, b, neg1))

        def cond(m):
            return m > PROB_THRESH

        def body(m):
            def fb(i, acc):
                eq = cm_v[pl.ds(pl.multiple_of(i * L, L), L)] == m
                cand = jnp.where(eq, lane_iota + i * L, BIG_I32)
                return jnp.minimum(acc, cand)
            ci = jnp.min(lax.fori_loop(0, CM_VREGS, fb,
                                       jnp.full((L,), BIG_I32, jnp.int32)))
            wv = w_v[pl.ds(pl.multiple_of(ci * L, L), L)]
            lane = jnp.min(jnp.where(wv == m, lane_iota, BIG_I32))
            j = ci * L + lane
            jv = jnp.full((L,), j, jnp.int32)

            ly1 = plsc.load_gather(cy1_v, [jv])
            lx1 = plsc.load_gather(cx1_v, [jv])
            ly2 = plsc.load_gather(cy2_v, [jv])
            lx2 = plsc.load_gather(cx2_v, [jv])
            lidx = plsc.load_gather(cidx_v, [jv])
            larea = (ly2 - ly1) * (lx2 - lx1)
            mv = jnp.full((L,), m, jnp.float32)

            plsc.store_scatter(o0_v, [lidx], ly1, mask=lane0)
            plsc.store_scatter(o1_v, [lidx], lx1, mask=lane0)
            plsc.store_scatter(o2_v, [lidx], ly2, mask=lane0)
            plsc.store_scatter(o3_v, [lidx], lx2, mask=lane0)
            plsc.store_scatter(o4_v, [lidx], mv, mask=lane0)

            def sb(i, carry):
                d = pl.ds(pl.multiple_of(i * L, L), L)
                w = w_v[d]
                y1 = cy1_v[d]
                x1 = cx1_v[d]
                y2 = cy2_v[d]
                x2 = cx2_v[d]
                ty = jnp.maximum(y1, ly1)
                tx = jnp.maximum(x1, lx1)
                by = jnp.minimum(y2, ly2)
                bx = jnp.minimum(x2, lx2)
                inter = jnp.maximum(by - ty, 0.0) * jnp.maximum(bx - tx, 0.0)
                union = (y2 - y1) * (x2 - x1) + larea - inter
                sup = inter > NMS_THRESH * union
                nw = jnp.where(sup, -1.0, w)
                w_v[d] = nw
                plsc.store_scatter(cm_v, [jnp.full((L,), i, jnp.int32)],
                                   jnp.full((L,), jnp.max(nw), jnp.float32),
                                   mask=lane0)
                return carry
            lax.fori_loop(0, nch, sb, 0)
            return global_max()

        lax.while_loop(cond, body, global_max())

        pltpu.sync_copy(o0_v, out_hbm.at[wid * 5 + 0])
        pltpu.sync_copy(o1_v, out_hbm.at[wid * 5 + 1])
        pltpu.sync_copy(o2_v, out_hbm.at[wid * 5 + 2])
        pltpu.sync_copy(o3_v, out_hbm.at[wid * 5 + 3])
        pltpu.sync_copy(o4_v, out_hbm.at[wid * 5 + 4])


_sc_nms = functools.partial(
    pl.kernel,
    out_type=jax.ShapeDtypeStruct((NFG * 5, NP), jnp.float32),
    mesh=_mesh,
    scratch_types=[
        pltpu.VMEM((NP,), jnp.float32),
        pltpu.VMEM((NP,), jnp.float32),
        pltpu.VMEM((NP,), jnp.float32),
        pltpu.VMEM((NP,), jnp.float32),
        pltpu.VMEM((NP,), jnp.float32),
        pltpu.VMEM((NP,), jnp.float32),
        pltpu.VMEM((NP,), jnp.float32),
        pltpu.VMEM((NP,), jnp.float32),
        pltpu.VMEM((NP,), jnp.float32),
        pltpu.VMEM((NP,), jnp.float32),
        pltpu.VMEM((NP,), jnp.int32),
        pltpu.VMEM((CM_CAP,), jnp.float32),
        pltpu.VMEM((NP,), jnp.float32),
        pltpu.VMEM((NP,), jnp.float32),
        pltpu.VMEM((NP,), jnp.float32),
        pltpu.VMEM((NP,), jnp.float32),
        pltpu.VMEM((NP,), jnp.float32),
    ],
    compiler_params=pltpu.CompilerParams(needs_layout_passes=False),
)(_nms_body)


def kernel(raw_cls_bbox, raw_prob):
    cls_bbox = raw_cls_bbox.reshape(N, NCLS, 4)
    boxes_t = jnp.transpose(cls_bbox[:, 1:, :], (1, 2, 0))
    boxes_p = jnp.concatenate(
        [boxes_t, jnp.zeros((NFG, 4, NP - N), jnp.float32)],
        axis=-1).reshape(NFG * 4, NP)
    scores_t = jnp.transpose(raw_prob[:, 1:], (1, 0))
    scores_p = jnp.concatenate(
        [scores_t, jnp.full((NFG, NP - N), -1.0, jnp.float32)], axis=-1)
    out = _sc_nms(scores_p, boxes_p).reshape(NFG, 5, NP)
    return jnp.transpose(out[:, :, :N], (0, 2, 1))


# speedup vs baseline: 346.0239x; 4.4730x over previous
import functools

import jax
import jax.numpy as jnp
from jax import lax
from jax.experimental import pallas as pl
from jax.experimental.pallas import tpu as pltpu
from jax.experimental.pallas import tpu_sc as plsc

N = 5000
NCLS = 21
NFG = NCLS - 1
PROB_THRESH = 0.05
NMS_THRESH = 0.3
L = 16
NP = 5024
NCHUNKS = NP // L
NUM_CORES = 2
BIG_I32 = 2**30
T = 4
R = 16

_mesh = plsc.VectorSubcoreMesh(core_axis_name="c", subcore_axis_name="s")


def _nms_body(scores_hbm, boxes_hbm, out_hbm,
              s_v, y1_v, x1_v, y2_v, x2_v,
              w_v, cy1_v, cx1_v, cy2_v, cx2_v, cidx_v, cm_v,
              o0_v, o1_v, o2_v, o3_v, o4_v, sem):
    wid = lax.axis_index("s") * NUM_CORES + lax.axis_index("c")

    @pl.when(wid < NFG)
    def _run():
        lane_iota = lax.iota(jnp.int32, L)
        lane0 = lane_iota == 0
        neg1 = jnp.full((L,), -1.0, jnp.float32)
        zero16 = jnp.zeros((L,), jnp.float32)

        cps = [
            pltpu.async_copy(scores_hbm.at[wid], s_v, sem),
            pltpu.async_copy(boxes_hbm.at[wid * 4 + 0], y1_v, sem),
            pltpu.async_copy(boxes_hbm.at[wid * 4 + 1], x1_v, sem),
            pltpu.async_copy(boxes_hbm.at[wid * 4 + 2], y2_v, sem),
            pltpu.async_copy(boxes_hbm.at[wid * 4 + 3], x2_v, sem),
        ]

        def init_body(i, carry):
            d = pl.ds(pl.multiple_of(i * L, L), L)
            w_v[d] = neg1
            o0_v[d] = zero16
            o1_v[d] = zero16
            o2_v[d] = zero16
            o3_v[d] = zero16
            o4_v[d] = zero16
            return carry
        lax.fori_loop(0, NCHUNKS, init_body, 0)

        def cm_init(i, carry):
            cm_v[pl.ds(pl.multiple_of(i * L, L), L)] = neg1
            return carry
        lax.fori_loop(0, NCHUNKS // L + 1, cm_init, 0)

        for cp in cps:
            cp.wait()

        def compact_body(i, base):
            d = pl.ds(pl.multiple_of(i * L, L), L)
            sv = s_v[d]
            msk = sv > PROB_THRESH
            pos = plsc.cumsum(msk.astype(jnp.int32))
            cnt = jnp.max(pos)
            dest = pos + (base - 1)
            plsc.store_scatter(w_v, [dest], sv, mask=msk)
            plsc.store_scatter(cy1_v, [dest], y1_v[d], mask=msk)
            plsc.store_scatter(cx1_v, [dest], x1_v[d], mask=msk)
            plsc.store_scatter(cy2_v, [dest], y2_v[d], mask=msk)
            plsc.store_scatter(cx2_v, [dest], x2_v[d], mask=msk)
            plsc.store_scatter(cidx_v, [dest], lane_iota + i * L, mask=msk)
            return base + cnt
        m_count = lax.fori_loop(0, NCHUNKS, compact_body, 0)
        nch0 = (m_count + (L - 1)) // L

        def cm_seed(i, carry):
            wv = w_v[pl.ds(pl.multiple_of(i * L, L), L)]
            plsc.store_scatter(cm_v, [jnp.full((L,), i, jnp.int32)],
                               jnp.full((L,), jnp.max(wv), jnp.float32),
                               mask=lane0)
            return carry
        lax.fori_loop(0, nch0, cm_seed, 0)

        def global_max(ncv):
            def b(i, acc):
                return jnp.maximum(acc, cm_v[pl.ds(pl.multiple_of(i * L, L), L)])
            return jnp.max(lax.fori_loop(0, ncv, b, neg1))

        def locate(m_k, ncv):
            def fb(i, acc):
                eq = cm_v[pl.ds(pl.multiple_of(i * L, L), L)] == m_k
                cand = jnp.where(eq, lane_iota + i * L, BIG_I32)
                return jnp.minimum(acc, cand)
            ci = jnp.min(lax.fori_loop(0, ncv, fb,
                                       jnp.full((L,), BIG_I32, jnp.int32)))
            wv = w_v[pl.ds(pl.multiple_of(ci * L, L), L)]
            lane = jnp.min(jnp.where(wv == m_k, lane_iota, BIG_I32))
            return ci, wv, lane

        def round_cond(carry):
            return carry[0] > PROB_THRESH

        def round_body(carry):
            m0, nch_old = carry

            def rc_body(i, base):
                d = pl.ds(pl.multiple_of(i * L, L), L)
                wv = w_v[d]
                v1 = cy1_v[d]
                v2 = cx1_v[d]
                v3 = cy2_v[d]
                v4 = cx2_v[d]
                vi = cidx_v[d]
                msk = wv > PROB_THRESH
                pos = plsc.cumsum(msk.astype(jnp.int32))
                cnt = jnp.max(pos)
                dest = pos + (base - 1)
                plsc.store_scatter(w_v, [dest], wv, mask=msk)
                plsc.store_scatter(cy1_v, [dest], v1, mask=msk)
                plsc.store_scatter(cx1_v, [dest], v2, mask=msk)
                plsc.store_scatter(cy2_v, [dest], v3, mask=msk)
                plsc.store_scatter(cx2_v, [dest], v4, mask=msk)
                plsc.store_scatter(cidx_v, [dest], vi, mask=msk)
                return base + cnt
            alive = lax.fori_loop(0, nch_old, rc_body, 0)
            nch = (alive + (L - 1)) // L

            def cl_body(i, carry2):
                d = pl.ds(pl.multiple_of(i * L, L), L)
                wv = jnp.where(lane_iota + i * L < alive, w_v[d], -1.0)
                w_v[d] = wv
                plsc.store_scatter(cm_v, [jnp.full((L,), i, jnp.int32)],
                                   jnp.full((L,), jnp.max(wv), jnp.float32),
                                   mask=lane0)
                return carry2
            lax.fori_loop(0, nch_old, cl_body, 0)

            ncv = (nch + (L - 1)) // L

            def iter_body(_, m):
                @pl.when(m > PROB_THRESH)
                def _it():
                    lead = []
                    for k in range(T):
                        m_k = m if k == 0 else global_max(ncv)
                        ci, wv, lane = locate(m_k, ncv)
                        j = ci * L + lane
                        jv = jnp.full((L,), j, jnp.int32)
                        ly1 = plsc.load_gather(cy1_v, [jv])
                        lx1 = plsc.load_gather(cx1_v, [jv])
                        ly2 = plsc.load_gather(cy2_v, [jv])
                        lx2 = plsc.load_gather(cx2_v, [jv])
                        lidx = plsc.load_gather(cidx_v, [jv])
                        nw = jnp.where(lane_iota == lane, -1.0, wv)
                        w_v[pl.ds(pl.multiple_of(ci * L, L), L)] = nw
                        plsc.store_scatter(
                            cm_v, [jnp.full((L,), ci, jnp.int32)],
                            jnp.full((L,), jnp.max(nw), jnp.float32),
                            mask=lane0)
                        valid = jnp.full((L,), m_k > PROB_THRESH)
                        lead.append((ly1, lx1, ly2, lx2, lidx, m_k, valid))

                    def ovl(a, b):
                        ay1, ax1, ay2, ax2 = a[0], a[1], a[2], a[3]
                        by1, bx1, by2, bx2 = b[0], b[1], b[2], b[3]
                        ih = jnp.maximum(
                            jnp.minimum(ay2, by2) - jnp.maximum(ay1, by1), 0.0)
                        iw = jnp.maximum(
                            jnp.minimum(ax2, bx2) - jnp.maximum(ax1, bx1), 0.0)
                        inter = ih * iw
                        aa = (ay2 - ay1) * (ax2 - ax1)
                        ab = (by2 - by1) * (bx2 - bx1)
                        return inter > NMS_THRESH * (aa + ab - inter)

                    acc = []
                    for k in range(T):
                        a_k = lead[k][6]
                        for kk in range(k):
                            a_k = a_k & ~(acc[kk] & ovl(lead[kk], lead[k]))
                        acc.append(a_k)

                    eff = []
                    for k in range(T):
                        ly1, lx1, ly2, lx2, lidx, m_k, _ = lead[k]
                        mk = lane0 & acc[k]
                        plsc.store_scatter(o0_v, [lidx], ly1, mask=mk)
                        plsc.store_scatter(o1_v, [lidx], lx1, mask=mk)
                        plsc.store_scatter(o2_v, [lidx], ly2, mask=mk)
                        plsc.store_scatter(o3_v, [lidx], lx2, mask=mk)
                        plsc.store_scatter(o4_v, [lidx],
                                           jnp.full((L,), m_k, jnp.float32),
                                           mask=mk)
                        eff.append((jnp.where(acc[k], ly1, 0.0),
                                    jnp.where(acc[k], lx1, 0.0),
                                    jnp.where(acc[k], ly2, 0.0),
                                    jnp.where(acc[k], lx2, 0.0)))

                    la = [(e[2] - e[0]) * (e[3] - e[1]) for e in eff]

                    def sb(i, carry2):
                        d = pl.ds(pl.multiple_of(i * L, L), L)
                        w = w_v[d]
                        y1 = cy1_v[d]
                        x1 = cx1_v[d]
                        y2 = cy2_v[d]
                        x2 = cx2_v[d]
                        area = (y2 - y1) * (x2 - x1)
                        sup = None
                        for k in range(T):
                            ey1, ex1, ey2, ex2 = eff[k]
                            ih = jnp.maximum(jnp.minimum(y2, ey2)
                                             - jnp.maximum(y1, ey1), 0.0)
                            iw = jnp.maximum(jnp.minimum(x2, ex2)
                                             - jnp.maximum(x1, ex1), 0.0)
                            inter = ih * iw
                            s_k = inter > NMS_THRESH * (area + la[k] - inter)
                            sup = s_k if sup is None else (sup | s_k)
                        nw = jnp.where(sup, -1.0, w)
                        w_v[d] = nw
                        plsc.store_scatter(
                            cm_v, [jnp.full((L,), i, jnp.int32)],
                            jnp.full((L,), jnp.max(nw), jnp.float32),
                            mask=lane0)
                        return carry2
                    lax.fori_loop(0, nch, sb, 0)
                return global_max(ncv)

            m_out = lax.fori_loop(0, R, iter_body, m0)
            return (m_out, nch)

        lax.while_loop(round_cond, round_body, (global_max((nch0 + L - 1) // L),
                                                nch0))

        pltpu.sync_copy(o0_v, out_hbm.at[wid * 5 + 0])
        pltpu.sync_copy(o1_v, out_hbm.at[wid * 5 + 1])
        pltpu.sync_copy(o2_v, out_hbm.at[wid * 5 + 2])
        pltpu.sync_copy(o3_v, out_hbm.at[wid * 5 + 3])
        pltpu.sync_copy(o4_v, out_hbm.at[wid * 5 + 4])


_sc_nms = functools.partial(
    pl.kernel,
    out_type=jax.ShapeDtypeStruct((NFG * 5, NP), jnp.float32),
    mesh=_mesh,
    scratch_types=[
        pltpu.VMEM((NP,), jnp.float32),
        pltpu.VMEM((NP,), jnp.float32),
        pltpu.VMEM((NP,), jnp.float32),
        pltpu.VMEM((NP,), jnp.float32),
        pltpu.VMEM((NP,), jnp.float32),
        pltpu.VMEM((NP,), jnp.float32),
        pltpu.VMEM((NP,), jnp.float32),
        pltpu.VMEM((NP,), jnp.float32),
        pltpu.VMEM((NP,), jnp.float32),
        pltpu.VMEM((NP,), jnp.float32),
        pltpu.VMEM((NP,), jnp.int32),
        pltpu.VMEM((320,), jnp.float32),
        pltpu.VMEM((NP,), jnp.float32),
        pltpu.VMEM((NP,), jnp.float32),
        pltpu.VMEM((NP,), jnp.float32),
        pltpu.VMEM((NP,), jnp.float32),
        pltpu.VMEM((NP,), jnp.float32),
        pltpu.SemaphoreType.DMA,
    ],
    compiler_params=pltpu.CompilerParams(needs_layout_passes=False),
)(_nms_body)


def kernel(raw_cls_bbox, raw_prob):
    cls_bbox = raw_cls_bbox.reshape(N, NCLS, 4)
    boxes_t = jnp.transpose(cls_bbox[:, 1:, :], (1, 2, 0))
    boxes_p = jnp.concatenate(
        [boxes_t, jnp.zeros((NFG, 4, NP - N), jnp.float32)],
        axis=-1).reshape(NFG * 4, NP)
    scores_t = jnp.transpose(raw_prob[:, 1:], (1, 0))
    scores_p = jnp.concatenate(
        [scores_t, jnp.full((NFG, NP - N), -1.0, jnp.float32)], axis=-1)
    out = _sc_nms(scores_p, boxes_p).reshape(NFG, 5, NP)
    return jnp.transpose(out[:, :, :N], (0, 2, 1))


# speedup vs baseline: 443.9419x; 1.2830x over previous
import functools

import jax
import jax.numpy as jnp
from jax import lax
from jax.experimental import pallas as pl
from jax.experimental.pallas import tpu as pltpu
from jax.experimental.pallas import tpu_sc as plsc

N = 5000
NCLS = 21
NFG = NCLS - 1
PROB_THRESH = 0.05
NMS_THRESH = 0.3
L = 16
NP = 5024
NCHUNKS = NP // L
NUM_CORES = 2
BIG_I32 = 2**30
T = 4
R = 16

_mesh = plsc.VectorSubcoreMesh(core_axis_name="c", subcore_axis_name="s")


def _nms_body(scores_hbm, boxes_hbm, out_hbm,
              s_v, y1_v, x1_v, y2_v, x2_v,
              w_v, cy1_v, cx1_v, cy2_v, cx2_v, cidx_v, cm_v,
              o0_v, o1_v, o2_v, o3_v, o4_v, sem):
    wid = lax.axis_index("s") * NUM_CORES + lax.axis_index("c")

    @pl.when(wid < NFG)
    def _run():
        lane_iota = lax.iota(jnp.int32, L)
        lane0 = lane_iota == 0
        neg1 = jnp.full((L,), -1.0, jnp.float32)
        zero16 = jnp.zeros((L,), jnp.float32)

        cps = [
            pltpu.async_copy(scores_hbm.at[wid], s_v, sem),
            pltpu.async_copy(boxes_hbm.at[wid * 4 + 0], y1_v, sem),
            pltpu.async_copy(boxes_hbm.at[wid * 4 + 1], x1_v, sem),
            pltpu.async_copy(boxes_hbm.at[wid * 4 + 2], y2_v, sem),
            pltpu.async_copy(boxes_hbm.at[wid * 4 + 3], x2_v, sem),
        ]

        @plsc.parallel_loop(0, NCHUNKS, 1, unroll=4)
        def init_body(i):
            d = pl.ds(pl.multiple_of(i * L, L), L)
            w_v[d] = neg1
            o0_v[d] = zero16
            o1_v[d] = zero16
            o2_v[d] = zero16
            o3_v[d] = zero16
            o4_v[d] = zero16

        def cm_init(i, carry):
            cm_v[pl.ds(pl.multiple_of(i * L, L), L)] = neg1
            return carry
        lax.fori_loop(0, NCHUNKS // L + 1, cm_init, 0)

        for cp in cps:
            cp.wait()

        def compact_body(i, base):
            d = pl.ds(pl.multiple_of(i * L, L), L)
            sv = s_v[d]
            msk = sv > PROB_THRESH
            pos = plsc.cumsum(msk.astype(jnp.int32))
            cnt = jnp.max(pos)
            dest = pos + (base - 1)
            plsc.store_scatter(w_v, [dest], sv, mask=msk)
            plsc.store_scatter(cy1_v, [dest], y1_v[d], mask=msk)
            plsc.store_scatter(cx1_v, [dest], x1_v[d], mask=msk)
            plsc.store_scatter(cy2_v, [dest], y2_v[d], mask=msk)
            plsc.store_scatter(cx2_v, [dest], x2_v[d], mask=msk)
            plsc.store_scatter(cidx_v, [dest], lane_iota + i * L, mask=msk)
            return base + cnt
        m_count = lax.fori_loop(0, NCHUNKS, compact_body, 0)
        nch0 = (m_count + (L - 1)) // L

        @plsc.parallel_loop(0, nch0, 1, unroll=4)
        def cm_seed(i):
            wv = w_v[pl.ds(pl.multiple_of(i * L, L), L)]
            plsc.store_scatter(cm_v, [jnp.full((L,), i, jnp.int32)],
                               jnp.full((L,), jnp.max(wv), jnp.float32),
                               mask=lane0)

        def global_max(ncv):
            def b(i, acc):
                return jnp.maximum(acc, cm_v[pl.ds(pl.multiple_of(i * L, L), L)])
            return jnp.max(lax.fori_loop(0, ncv, b, neg1))

        def locate(m_k, ncv):
            def fb(i, acc):
                eq = cm_v[pl.ds(pl.multiple_of(i * L, L), L)] == m_k
                cand = jnp.where(eq, lane_iota + i * L, BIG_I32)
                return jnp.minimum(acc, cand)
            ci = jnp.min(lax.fori_loop(0, ncv, fb,
                                       jnp.full((L,), BIG_I32, jnp.int32)))
            wv = w_v[pl.ds(pl.multiple_of(ci * L, L), L)]
            lane = jnp.min(jnp.where(wv == m_k, lane_iota, BIG_I32))
            return ci, wv, lane

        def round_cond(carry):
            return carry[0] > PROB_THRESH

        def round_body(carry):
            m0, nch_old = carry

            def rc_body(i, base):
                d = pl.ds(pl.multiple_of(i * L, L), L)
                wv = w_v[d]
                v1 = cy1_v[d]
                v2 = cx1_v[d]
                v3 = cy2_v[d]
                v4 = cx2_v[d]
                vi = cidx_v[d]
                msk = wv > PROB_THRESH
                pos = plsc.cumsum(msk.astype(jnp.int32))
                cnt = jnp.max(pos)
                dest = pos + (base - 1)
                plsc.store_scatter(w_v, [dest], wv, mask=msk)
                plsc.store_scatter(cy1_v, [dest], v1, mask=msk)
                plsc.store_scatter(cx1_v, [dest], v2, mask=msk)
                plsc.store_scatter(cy2_v, [dest], v3, mask=msk)
                plsc.store_scatter(cx2_v, [dest], v4, mask=msk)
                plsc.store_scatter(cidx_v, [dest], vi, mask=msk)
                return base + cnt
            alive = lax.fori_loop(0, nch_old, rc_body, 0)
            nch = (alive + (L - 1)) // L

            @plsc.parallel_loop(0, nch_old, 1, unroll=4)
            def cl_body(i):
                d = pl.ds(pl.multiple_of(i * L, L), L)
                wv = jnp.where(lane_iota + i * L < alive, w_v[d], -1.0)
                w_v[d] = wv
                plsc.store_scatter(cm_v, [jnp.full((L,), i, jnp.int32)],
                                   jnp.full((L,), jnp.max(wv), jnp.float32),
                                   mask=lane0)

            ncv = (nch + (L - 1)) // L

            def iter_body(_, m):
                @pl.when(m > PROB_THRESH)
                def _it():
                    lead = []
                    for k in range(T):
                        m_k = m if k == 0 else global_max(ncv)
                        ci, wv, lane = locate(m_k, ncv)
                        j = ci * L + lane
                        jv = jnp.full((L,), j, jnp.int32)
                        ly1 = plsc.load_gather(cy1_v, [jv])
                        lx1 = plsc.load_gather(cx1_v, [jv])
                        ly2 = plsc.load_gather(cy2_v, [jv])
                        lx2 = plsc.load_gather(cx2_v, [jv])
                        lidx = plsc.load_gather(cidx_v, [jv])
                        nw = jnp.where(lane_iota == lane, -1.0, wv)
                        w_v[pl.ds(pl.multiple_of(ci * L, L), L)] = nw
                        plsc.store_scatter(
                            cm_v, [jnp.full((L,), ci, jnp.int32)],
                            jnp.full((L,), jnp.max(nw), jnp.float32),
                            mask=lane0)
                        valid = jnp.full((L,), m_k > PROB_THRESH)
                        lead.append((ly1, lx1, ly2, lx2, lidx, m_k, valid))

                    def ovl(a, b):
                        ay1, ax1, ay2, ax2 = a[0], a[1], a[2], a[3]
                        by1, bx1, by2, bx2 = b[0], b[1], b[2], b[3]
                        ih = jnp.maximum(
                            jnp.minimum(ay2, by2) - jnp.maximum(ay1, by1), 0.0)
                        iw = jnp.maximum(
                            jnp.minimum(ax2, bx2) - jnp.maximum(ax1, bx1), 0.0)
                        inter = ih * iw
                        aa = (ay2 - ay1) * (ax2 - ax1)
                        ab = (by2 - by1) * (bx2 - bx1)
                        return inter > NMS_THRESH * (aa + ab - inter)

                    acc = []
                    for k in range(T):
                        a_k = lead[k][6]
                        for kk in range(k):
                            a_k = a_k & ~(acc[kk] & ovl(lead[kk], lead[k]))
                        acc.append(a_k)

                    eff = []
                    for k in range(T):
                        ly1, lx1, ly2, lx2, lidx, m_k, _ = lead[k]
                        mk = lane0 & acc[k]
                        plsc.store_scatter(o0_v, [lidx], ly1, mask=mk)
                        plsc.store_scatter(o1_v, [lidx], lx1, mask=mk)
                        plsc.store_scatter(o2_v, [lidx], ly2, mask=mk)
                        plsc.store_scatter(o3_v, [lidx], lx2, mask=mk)
                        plsc.store_scatter(o4_v, [lidx],
                                           jnp.full((L,), m_k, jnp.float32),
                                           mask=mk)
                        eff.append((jnp.where(acc[k], ly1, 0.0),
                                    jnp.where(acc[k], lx1, 0.0),
                                    jnp.where(acc[k], ly2, 0.0),
                                    jnp.where(acc[k], lx2, 0.0)))

                    la = [(e[2] - e[0]) * (e[3] - e[1]) for e in eff]

                    @plsc.parallel_loop(0, nch, 1, unroll=4)
                    def sb(i):
                        d = pl.ds(pl.multiple_of(i * L, L), L)
                        w = w_v[d]
                        y1 = cy1_v[d]
                        x1 = cx1_v[d]
                        y2 = cy2_v[d]
                        x2 = cx2_v[d]
                        area = (y2 - y1) * (x2 - x1)
                        sup = None
                        for k in range(T):
                            ey1, ex1, ey2, ex2 = eff[k]
                            ih = jnp.maximum(jnp.minimum(y2, ey2)
                                             - jnp.maximum(y1, ey1), 0.0)
                            iw = jnp.maximum(jnp.minimum(x2, ex2)
                                             - jnp.maximum(x1, ex1), 0.0)
                            inter = ih * iw
                            s_k = inter > NMS_THRESH * (area + la[k] - inter)
                            sup = s_k if sup is None else (sup | s_k)
                        nw = jnp.where(sup, -1.0, w)
                        w_v[d] = nw
                        plsc.store_scatter(
                            cm_v, [jnp.full((L,), i, jnp.int32)],
                            jnp.full((L,), jnp.max(nw), jnp.float32),
                            mask=lane0)
                return global_max(ncv)

            m_out = lax.fori_loop(0, R, iter_body, m0)
            return (m_out, nch)

        lax.while_loop(round_cond, round_body, (global_max((nch0 + L - 1) // L),
                                                nch0))

        pltpu.sync_copy(o0_v, out_hbm.at[wid * 5 + 0])
        pltpu.sync_copy(o1_v, out_hbm.at[wid * 5 + 1])
        pltpu.sync_copy(o2_v, out_hbm.at[wid * 5 + 2])
        pltpu.sync_copy(o3_v, out_hbm.at[wid * 5 + 3])
        pltpu.sync_copy(o4_v, out_hbm.at[wid * 5 + 4])


_sc_nms = functools.partial(
    pl.kernel,
    out_type=jax.ShapeDtypeStruct((NFG * 5, NP), jnp.float32),
    mesh=_mesh,
    scratch_types=[
        pltpu.VMEM((NP,), jnp.float32),
        pltpu.VMEM((NP,), jnp.float32),
        pltpu.VMEM((NP,), jnp.float32),
        pltpu.VMEM((NP,), jnp.float32),
        pltpu.VMEM((NP,), jnp.float32),
        pltpu.VMEM((NP,), jnp.float32),
        pltpu.VMEM((NP,), jnp.float32),
        pltpu.VMEM((NP,), jnp.float32),
        pltpu.VMEM((NP,), jnp.float32),
        pltpu.VMEM((NP,), jnp.float32),
        pltpu.VMEM((NP,), jnp.int32),
        pltpu.VMEM((320,), jnp.float32),
        pltpu.VMEM((NP,), jnp.float32),
        pltpu.VMEM((NP,), jnp.float32),
        pltpu.VMEM((NP,), jnp.float32),
        pltpu.VMEM((NP,), jnp.float32),
        pltpu.VMEM((NP,), jnp.float32),
        pltpu.SemaphoreType.DMA,
    ],
    compiler_params=pltpu.CompilerParams(needs_layout_passes=False),
)(_nms_body)


def kernel(raw_cls_bbox, raw_prob):
    cls_bbox = raw_cls_bbox.reshape(N, NCLS, 4)
    boxes_t = jnp.transpose(cls_bbox[:, 1:, :], (1, 2, 0))
    boxes_p = jnp.concatenate(
        [boxes_t, jnp.zeros((NFG, 4, NP - N), jnp.float32)],
        axis=-1).reshape(NFG * 4, NP)
    scores_t = jnp.transpose(raw_prob[:, 1:], (1, 0))
    scores_p = jnp.concatenate(
        [scores_t, jnp.full((NFG, NP - N), -1.0, jnp.float32)], axis=-1)
    out = _sc_nms(scores_p, boxes_p).reshape(NFG, 5, NP)
    return jnp.transpose(out[:, :, :N], (0, 2, 1))


# speedup vs baseline: 464.4766x; 1.0463x over previous
import functools

import jax
import jax.numpy as jnp
from jax import lax
from jax.experimental import pallas as pl
from jax.experimental.pallas import tpu as pltpu
from jax.experimental.pallas import tpu_sc as plsc

N = 5000
NCLS = 21
NFG = NCLS - 1
PROB_THRESH = 0.05
NMS_THRESH = 0.3
L = 16
NP = 5024
NCHUNKS = NP // L
NUM_CORES = 2
BIG_I32 = 2**30
T = 4
R = 16

_mesh = plsc.VectorSubcoreMesh(core_axis_name="c", subcore_axis_name="s")


def _nms_body(scores_hbm, boxes_hbm, out_hbm,
              s_v, y1_v, x1_v, y2_v, x2_v,
              w_v, cy1_v, cx1_v, cy2_v, cx2_v, cidx_v, cm_v,
              o0_v, o1_v, o2_v, o3_v, o4_v, sem):
    wid = lax.axis_index("s") * NUM_CORES + lax.axis_index("c")

    @pl.when(wid < NFG)
    def _run():
        lane_iota = lax.iota(jnp.int32, L)
        lane0 = lane_iota == 0
        neg1 = jnp.full((L,), -1.0, jnp.float32)
        zero16 = jnp.zeros((L,), jnp.float32)

        cps = [
            pltpu.async_copy(scores_hbm.at[wid], s_v, sem),
            pltpu.async_copy(boxes_hbm.at[wid * 4 + 0], y1_v, sem),
            pltpu.async_copy(boxes_hbm.at[wid * 4 + 1], x1_v, sem),
            pltpu.async_copy(boxes_hbm.at[wid * 4 + 2], y2_v, sem),
            pltpu.async_copy(boxes_hbm.at[wid * 4 + 3], x2_v, sem),
        ]

        @plsc.parallel_loop(0, NCHUNKS, 1, unroll=4)
        def init_body(i):
            d = pl.ds(pl.multiple_of(i * L, L), L)
            w_v[d] = neg1
            o0_v[d] = zero16
            o1_v[d] = zero16
            o2_v[d] = zero16
            o3_v[d] = zero16
            o4_v[d] = zero16

        def cm_init(i, carry):
            cm_v[pl.ds(pl.multiple_of(i * L, L), L)] = neg1
            return carry
        lax.fori_loop(0, NCHUNKS // L + 1, cm_init, 0)

        for cp in cps:
            cp.wait()

        def compact_body(i, base):
            d = pl.ds(pl.multiple_of(i * L, L), L)
            sv = s_v[d]
            msk = sv > PROB_THRESH
            pos = plsc.cumsum(msk.astype(jnp.int32))
            cnt = jnp.max(pos)
            dest = pos + (base - 1)
            plsc.store_scatter(w_v, [dest], sv, mask=msk)
            plsc.store_scatter(cy1_v, [dest], y1_v[d], mask=msk)
            plsc.store_scatter(cx1_v, [dest], x1_v[d], mask=msk)
            plsc.store_scatter(cy2_v, [dest], y2_v[d], mask=msk)
            plsc.store_scatter(cx2_v, [dest], x2_v[d], mask=msk)
            plsc.store_scatter(cidx_v, [dest], lane_iota + i * L, mask=msk)
            return base + cnt
        m_count = lax.fori_loop(0, NCHUNKS, compact_body, 0)
        nch0 = (m_count + (L - 1)) // L

        @plsc.parallel_loop(0, nch0, 1, unroll=4)
        def cm_seed(i):
            wv = w_v[pl.ds(pl.multiple_of(i * L, L), L)]
            plsc.store_scatter(cm_v, [jnp.full((L,), i, jnp.int32)],
                               jnp.full((L,), jnp.max(wv), jnp.float32),
                               mask=lane0)

        def global_max(ncv):
            def b(i, acc):
                return jnp.maximum(acc, cm_v[pl.ds(pl.multiple_of(i * L, L), L)])
            return jnp.max(lax.fori_loop(0, ncv, b, neg1))

        def max_locate(ncv):
            def fb(i, acc):
                acc_v, acc_p = acc
                v = cm_v[pl.ds(pl.multiple_of(i * L, L), L)]
                gt = v > acc_v
                return (jnp.maximum(acc_v, v),
                        jnp.where(gt, lane_iota + i * L, acc_p))
            acc_v, acc_p = lax.fori_loop(
                0, ncv, fb,
                (jnp.full((L,), -2.0, jnp.float32),
                 jnp.full((L,), BIG_I32, jnp.int32)))
            m_k = jnp.max(acc_v)
            ci = jnp.min(jnp.where(acc_v == m_k, acc_p, BIG_I32))
            wv = w_v[pl.ds(pl.multiple_of(ci * L, L), L)]
            lane = jnp.min(jnp.where(wv == m_k, lane_iota, BIG_I32))
            return m_k, ci, wv, lane

        def round_cond(carry):
            return carry[0] > PROB_THRESH

        def round_body(carry):
            m0, nch_old = carry

            def rc_body(i, base):
                d = pl.ds(pl.multiple_of(i * L, L), L)
                wv = w_v[d]
                v1 = cy1_v[d]
                v2 = cx1_v[d]
                v3 = cy2_v[d]
                v4 = cx2_v[d]
                vi = cidx_v[d]
                msk = wv > PROB_THRESH
                pos = plsc.cumsum(msk.astype(jnp.int32))
                cnt = jnp.max(pos)
                dest = pos + (base - 1)
                plsc.store_scatter(w_v, [dest], wv, mask=msk)
                plsc.store_scatter(cy1_v, [dest], v1, mask=msk)
                plsc.store_scatter(cx1_v, [dest], v2, mask=msk)
                plsc.store_scatter(cy2_v, [dest], v3, mask=msk)
                plsc.store_scatter(cx2_v, [dest], v4, mask=msk)
                plsc.store_scatter(cidx_v, [dest], vi, mask=msk)
                return base + cnt
            alive = lax.fori_loop(0, nch_old, rc_body, 0)
            nch = (alive + (L - 1)) // L

            @plsc.parallel_loop(0, nch_old, 1, unroll=4)
            def cl_body(i):
                d = pl.ds(pl.multiple_of(i * L, L), L)
                wv = jnp.where(lane_iota + i * L < alive, w_v[d], -1.0)
                w_v[d] = wv
                plsc.store_scatter(cm_v, [jnp.full((L,), i, jnp.int32)],
                                   jnp.full((L,), jnp.max(wv), jnp.float32),
                                   mask=lane0)

            ncv = (nch + (L - 1)) // L

            def iter_body(_, m):
                @pl.when(m > PROB_THRESH)
                def _it():
                    lead = []
                    for k in range(T):
                        m_k, ci, wv, lane = max_locate(ncv)
                        j = ci * L + lane
                        jv = jnp.full((L,), j, jnp.int32)
                        ly1 = plsc.load_gather(cy1_v, [jv])
                        lx1 = plsc.load_gather(cx1_v, [jv])
                        ly2 = plsc.load_gather(cy2_v, [jv])
                        lx2 = plsc.load_gather(cx2_v, [jv])
                        lidx = plsc.load_gather(cidx_v, [jv])
                        nw = jnp.where(lane_iota == lane, -1.0, wv)
                        w_v[pl.ds(pl.multiple_of(ci * L, L), L)] = nw
                        plsc.store_scatter(
                            cm_v, [jnp.full((L,), ci, jnp.int32)],
                            jnp.full((L,), jnp.max(nw), jnp.float32),
                            mask=lane0)
                        valid = jnp.full((L,), m_k > PROB_THRESH)
                        lead.append((ly1, lx1, ly2, lx2, lidx, m_k, valid))

                    def ovl(a, b):
                        ay1, ax1, ay2, ax2 = a[0], a[1], a[2], a[3]
                        by1, bx1, by2, bx2 = b[0], b[1], b[2], b[3]
                        ih = jnp.maximum(
                            jnp.minimum(ay2, by2) - jnp.maximum(ay1, by1), 0.0)
                        iw = jnp.maximum(
                            jnp.minimum(ax2, bx2) - jnp.maximum(ax1, bx1), 0.0)
                        inter = ih * iw
                        aa = (ay2 - ay1) * (ax2 - ax1)
                        ab = (by2 - by1) * (bx2 - bx1)
                        return inter > NMS_THRESH * (aa + ab - inter)

                    acc = []
                    for k in range(T):
                        a_k = lead[k][6]
                        for kk in range(k):
                            a_k = a_k & ~(acc[kk] & ovl(lead[kk], lead[k]))
                        acc.append(a_k)

                    eff = []
                    for k in range(T):
                        ly1, lx1, ly2, lx2, lidx, m_k, _ = lead[k]
                        mk = lane0 & acc[k]
                        plsc.store_scatter(o0_v, [lidx], ly1, mask=mk)
                        plsc.store_scatter(o1_v, [lidx], lx1, mask=mk)
                        plsc.store_scatter(o2_v, [lidx], ly2, mask=mk)
                        plsc.store_scatter(o3_v, [lidx], lx2, mask=mk)
                        plsc.store_scatter(o4_v, [lidx],
                                           jnp.full((L,), m_k, jnp.float32),
                                           mask=mk)
                        eff.append((jnp.where(acc[k], ly1, 0.0),
                                    jnp.where(acc[k], lx1, 0.0),
                                    jnp.where(acc[k], ly2, 0.0),
                                    jnp.where(acc[k], lx2, 0.0)))

                    la = [(e[2] - e[0]) * (e[3] - e[1]) for e in eff]

                    @plsc.parallel_loop(0, nch, 1, unroll=4)
                    def sb(i):
                        d = pl.ds(pl.multiple_of(i * L, L), L)
                        w = w_v[d]
                        y1 = cy1_v[d]
                        x1 = cx1_v[d]
                        y2 = cy2_v[d]
                        x2 = cx2_v[d]
                        area = (y2 - y1) * (x2 - x1)
                        sup = None
                        for k in range(T):
                            ey1, ex1, ey2, ex2 = eff[k]
                            ih = jnp.maximum(jnp.minimum(y2, ey2)
                                             - jnp.maximum(y1, ey1), 0.0)
                            iw = jnp.maximum(jnp.minimum(x2, ex2)
                                             - jnp.maximum(x1, ex1), 0.0)
                            inter = ih * iw
                            s_k = inter > NMS_THRESH * (area + la[k] - inter)
                            sup = s_k if sup is None else (sup | s_k)
                        nw = jnp.where(sup, -1.0, w)
                        w_v[d] = nw
                        plsc.store_scatter(
                            cm_v, [jnp.full((L,), i, jnp.int32)],
                            jnp.full((L,), jnp.max(nw), jnp.float32),
                            mask=lane0)
                return global_max(ncv)

            m_out = lax.fori_loop(0, R, iter_body, m0)
            return (m_out, nch)

        lax.while_loop(round_cond, round_body, (global_max((nch0 + L - 1) // L),
                                                nch0))

        pltpu.sync_copy(o0_v, out_hbm.at[wid * 5 + 0])
        pltpu.sync_copy(o1_v, out_hbm.at[wid * 5 + 1])
        pltpu.sync_copy(o2_v, out_hbm.at[wid * 5 + 2])
        pltpu.sync_copy(o3_v, out_hbm.at[wid * 5 + 3])
        pltpu.sync_copy(o4_v, out_hbm.at[wid * 5 + 4])


_sc_nms = functools.partial(
    pl.kernel,
    out_type=jax.ShapeDtypeStruct((NFG * 5, NP), jnp.float32),
    mesh=_mesh,
    scratch_types=[
        pltpu.VMEM((NP,), jnp.float32),
        pltpu.VMEM((NP,), jnp.float32),
        pltpu.VMEM((NP,), jnp.float32),
        pltpu.VMEM((NP,), jnp.float32),
        pltpu.VMEM((NP,), jnp.float32),
        pltpu.VMEM((NP,), jnp.float32),
        pltpu.VMEM((NP,), jnp.float32),
        pltpu.VMEM((NP,), jnp.float32),
        pltpu.VMEM((NP,), jnp.float32),
        pltpu.VMEM((NP,), jnp.float32),
        pltpu.VMEM((NP,), jnp.int32),
        pltpu.VMEM((320,), jnp.float32),
        pltpu.VMEM((NP,), jnp.float32),
        pltpu.VMEM((NP,), jnp.float32),
        pltpu.VMEM((NP,), jnp.float32),
        pltpu.VMEM((NP,), jnp.float32),
        pltpu.VMEM((NP,), jnp.float32),
        pltpu.SemaphoreType.DMA,
    ],
    compiler_params=pltpu.CompilerParams(needs_layout_passes=False),
)(_nms_body)


def kernel(raw_cls_bbox, raw_prob):
    cls_bbox = raw_cls_bbox.reshape(N, NCLS, 4)
    boxes_t = jnp.transpose(cls_bbox[:, 1:, :], (1, 2, 0))
    boxes_p = jnp.concatenate(
        [boxes_t, jnp.zeros((NFG, 4, NP - N), jnp.float32)],
        axis=-1).reshape(NFG * 4, NP)
    scores_t = jnp.transpose(raw_prob[:, 1:], (1, 0))
    scores_p = jnp.concatenate(
        [scores_t, jnp.full((NFG, NP - N), -1.0, jnp.float32)], axis=-1)
    out = _sc_nms(scores_p, boxes_p).reshape(NFG, 5, NP)
    return jnp.transpose(out[:, :, :N], (0, 2, 1))


# speedup vs baseline: 495.1389x; 1.0660x over previous
import functools

import jax
import jax.numpy as jnp
from jax import lax
from jax.experimental import pallas as pl
from jax.experimental.pallas import tpu as pltpu
from jax.experimental.pallas import tpu_sc as plsc

N = 5000
NCLS = 21
NFG = NCLS - 1
PROB_THRESH = 0.05
NMS_THRESH = 0.3
L = 16
NP = 5024
NCHUNKS = NP // L
NUM_CORES = 2
BIG_I32 = 2**30
T = 4
R = 8
TFRAC = NMS_THRESH / (1.0 + NMS_THRESH)

_mesh = plsc.VectorSubcoreMesh(core_axis_name="c", subcore_axis_name="s")


def _nms_body(scores_hbm, boxes_hbm, out_hbm,
              s_v, y1_v, x1_v, y2_v, x2_v,
              w_v, cy1_v, cx1_v, cy2_v, cx2_v, cidx_v, ca_v, cm_v,
              o0_v, o1_v, o2_v, o3_v, o4_v, sem):
    wid = lax.axis_index("s") * NUM_CORES + lax.axis_index("c")

    @pl.when(wid < NFG)
    def _run():
        lane_iota = lax.iota(jnp.int32, L)
        lane0 = lane_iota == 0
        neg1 = jnp.full((L,), -1.0, jnp.float32)
        zero16 = jnp.zeros((L,), jnp.float32)

        cps = [
            pltpu.async_copy(scores_hbm.at[wid], s_v, sem),
            pltpu.async_copy(boxes_hbm.at[wid * 4 + 0], y1_v, sem),
            pltpu.async_copy(boxes_hbm.at[wid * 4 + 1], x1_v, sem),
            pltpu.async_copy(boxes_hbm.at[wid * 4 + 2], y2_v, sem),
            pltpu.async_copy(boxes_hbm.at[wid * 4 + 3], x2_v, sem),
        ]

        @plsc.parallel_loop(0, NCHUNKS, 1, unroll=4)
        def init_body(i):
            d = pl.ds(pl.multiple_of(i * L, L), L)
            w_v[d] = neg1
            o0_v[d] = zero16
            o1_v[d] = zero16
            o2_v[d] = zero16
            o3_v[d] = zero16
            o4_v[d] = zero16

        def cm_init(i, carry):
            cm_v[pl.ds(pl.multiple_of(i * L, L), L)] = neg1
            return carry
        lax.fori_loop(0, NCHUNKS // L + 1, cm_init, 0)

        for cp in cps:
            cp.wait()

        def compact_body(i, base):
            d = pl.ds(pl.multiple_of(i * L, L), L)
            sv = s_v[d]
            msk = sv > PROB_THRESH
            pos = plsc.cumsum(msk.astype(jnp.int32))
            cnt = jnp.max(pos)
            dest = pos + (base - 1)
            v1 = y1_v[d]
            v2 = x1_v[d]
            v3 = y2_v[d]
            v4 = x2_v[d]
            plsc.store_scatter(w_v, [dest], sv, mask=msk)
            plsc.store_scatter(cy1_v, [dest], v1, mask=msk)
            plsc.store_scatter(cx1_v, [dest], v2, mask=msk)
            plsc.store_scatter(cy2_v, [dest], v3, mask=msk)
            plsc.store_scatter(cx2_v, [dest], v4, mask=msk)
            plsc.store_scatter(cidx_v, [dest], lane_iota + i * L, mask=msk)
            plsc.store_scatter(ca_v, [dest],
                               TFRAC * ((v3 - v1) * (v4 - v2)), mask=msk)
            return base + cnt
        m_count = lax.fori_loop(0, NCHUNKS, compact_body, 0)
        nch0 = (m_count + (L - 1)) // L

        @plsc.parallel_loop(0, nch0, 1, unroll=4)
        def cm_seed(i):
            wv = w_v[pl.ds(pl.multiple_of(i * L, L), L)]
            plsc.store_scatter(cm_v, [jnp.full((L,), i, jnp.int32)],
                               jnp.full((L,), jnp.max(wv), jnp.float32),
                               mask=lane0)

        def global_max(ncv):
            def b(i, acc):
                return jnp.maximum(acc, cm_v[pl.ds(pl.multiple_of(i * L, L), L)])
            return jnp.max(lax.fori_loop(0, ncv, b, neg1))

        def max_locate(ncv):
            def fb(i, acc):
                acc_v, acc_p = acc
                v = cm_v[pl.ds(pl.multiple_of(i * L, L), L)]
                gt = v > acc_v
                return (jnp.maximum(acc_v, v),
                        jnp.where(gt, lane_iota + i * L, acc_p))
            acc_v, acc_p = lax.fori_loop(
                0, ncv, fb,
                (jnp.full((L,), -2.0, jnp.float32),
                 jnp.full((L,), BIG_I32, jnp.int32)))
            m_k = jnp.max(acc_v)
            ci = jnp.min(jnp.where(acc_v == m_k, acc_p, BIG_I32))
            wv = w_v[pl.ds(pl.multiple_of(ci * L, L), L)]
            lane = jnp.min(jnp.where(wv == m_k, lane_iota, BIG_I32))
            return m_k, ci, wv, lane

        def round_cond(carry):
            return carry[0] > PROB_THRESH

        def round_body(carry):
            m0, nch_old = carry

            def rc_body(i, base):
                d = pl.ds(pl.multiple_of(i * L, L), L)
                wv = w_v[d]
                v1 = cy1_v[d]
                v2 = cx1_v[d]
                v3 = cy2_v[d]
                v4 = cx2_v[d]
                vi = cidx_v[d]
                va = ca_v[d]
                msk = wv > PROB_THRESH
                pos = plsc.cumsum(msk.astype(jnp.int32))
                cnt = jnp.max(pos)
                dest = pos + (base - 1)
                plsc.store_scatter(w_v, [dest], wv, mask=msk)
                plsc.store_scatter(cy1_v, [dest], v1, mask=msk)
                plsc.store_scatter(cx1_v, [dest], v2, mask=msk)
                plsc.store_scatter(cy2_v, [dest], v3, mask=msk)
                plsc.store_scatter(cx2_v, [dest], v4, mask=msk)
                plsc.store_scatter(cidx_v, [dest], vi, mask=msk)
                plsc.store_scatter(ca_v, [dest], va, mask=msk)
                return base + cnt
            alive = lax.fori_loop(0, nch_old, rc_body, 0)
            nch = (alive + (L - 1)) // L

            @plsc.parallel_loop(0, nch_old, 1, unroll=4)
            def cl_body(i):
                d = pl.ds(pl.multiple_of(i * L, L), L)
                wv = jnp.where(lane_iota + i * L < alive, w_v[d], -1.0)
                w_v[d] = wv
                plsc.store_scatter(cm_v, [jnp.full((L,), i, jnp.int32)],
                                   jnp.full((L,), jnp.max(wv), jnp.float32),
                                   mask=lane0)

            ncv = (nch + (L - 1)) // L

            def iter_body(_, m):
                @pl.when(m > PROB_THRESH)
                def _it():
                    lead = []
                    for k in range(T):
                        m_k, ci, wv, lane = max_locate(ncv)
                        j = ci * L + lane
                        jv = jnp.full((L,), j, jnp.int32)
                        ly1 = plsc.load_gather(cy1_v, [jv])
                        lx1 = plsc.load_gather(cx1_v, [jv])
                        ly2 = plsc.load_gather(cy2_v, [jv])
                        lx2 = plsc.load_gather(cx2_v, [jv])
                        lidx = plsc.load_gather(cidx_v, [jv])
                        nw = jnp.where(lane_iota == lane, -1.0, wv)
                        w_v[pl.ds(pl.multiple_of(ci * L, L), L)] = nw
                        plsc.store_scatter(
                            cm_v, [jnp.full((L,), ci, jnp.int32)],
                            jnp.full((L,), jnp.max(nw), jnp.float32),
                            mask=lane0)
                        valid = jnp.full((L,), m_k > PROB_THRESH)
                        lead.append((ly1, lx1, ly2, lx2, lidx, m_k, valid))

                    def ovl(a, b):
                        ay1, ax1, ay2, ax2 = a[0], a[1], a[2], a[3]
                        by1, bx1, by2, bx2 = b[0], b[1], b[2], b[3]
                        ih = jnp.maximum(
                            jnp.minimum(ay2, by2) - jnp.maximum(ay1, by1), 0.0)
                        iw = jnp.maximum(
                            jnp.minimum(ax2, bx2) - jnp.maximum(ax1, bx1), 0.0)
                        inter = ih * iw
                        aa = (ay2 - ay1) * (ax2 - ax1)
                        ab = (by2 - by1) * (bx2 - bx1)
                        return inter > NMS_THRESH * (aa + ab - inter)

                    acc = []
                    for k in range(T):
                        a_k = lead[k][6]
                        for kk in range(k):
                            a_k = a_k & ~(acc[kk] & ovl(lead[kk], lead[k]))
                        acc.append(a_k)

                    eff = []
                    for k in range(T):
                        ly1, lx1, ly2, lx2, lidx, m_k, _ = lead[k]
                        mk = lane0 & acc[k]
                        plsc.store_scatter(o0_v, [lidx], ly1, mask=mk)
                        plsc.store_scatter(o1_v, [lidx], lx1, mask=mk)
                        plsc.store_scatter(o2_v, [lidx], ly2, mask=mk)
                        plsc.store_scatter(o3_v, [lidx], lx2, mask=mk)
                        plsc.store_scatter(o4_v, [lidx],
                                           jnp.full((L,), m_k, jnp.float32),
                                           mask=mk)
                        eff.append((jnp.where(acc[k], ly1, 0.0),
                                    jnp.where(acc[k], lx1, 0.0),
                                    jnp.where(acc[k], ly2, 0.0),
                                    jnp.where(acc[k], lx2, 0.0)))

                    lca = [TFRAC * (e[2] - e[0]) * (e[3] - e[1])
                           for e in eff]

                    @plsc.parallel_loop(0, nch, 1, unroll=4)
                    def sb(i):
                        d = pl.ds(pl.multiple_of(i * L, L), L)
                        w = w_v[d]
                        y1 = cy1_v[d]
                        x1 = cx1_v[d]
                        y2 = cy2_v[d]
                        x2 = cx2_v[d]
                        pa = ca_v[d]
                        sup = None
                        for k in range(T):
                            ey1, ex1, ey2, ex2 = eff[k]
                            ih = jnp.maximum(jnp.minimum(y2, ey2)
                                             - jnp.maximum(y1, ey1), 0.0)
                            iw = jnp.maximum(jnp.minimum(x2, ex2)
                                             - jnp.maximum(x1, ex1), 0.0)
                            inter = ih * iw
                            s_k = inter > pa + lca[k]
                            sup = s_k if sup is None else (sup | s_k)
                        nw = jnp.where(sup, -1.0, w)
                        w_v[d] = nw
                        plsc.store_scatter(
                            cm_v, [jnp.full((L,), i, jnp.int32)],
                            jnp.full((L,), jnp.max(nw), jnp.float32),
                            mask=lane0)
                return global_max(ncv)

            m_out = lax.fori_loop(0, R, iter_body, m0)
            return (m_out, nch)

        lax.while_loop(round_cond, round_body, (global_max((nch0 + L - 1) // L),
                                                nch0))

        pltpu.sync_copy(o0_v, out_hbm.at[wid * 5 + 0])
        pltpu.sync_copy(o1_v, out_hbm.at[wid * 5 + 1])
        pltpu.sync_copy(o2_v, out_hbm.at[wid * 5 + 2])
        pltpu.sync_copy(o3_v, out_hbm.at[wid * 5 + 3])
        pltpu.sync_copy(o4_v, out_hbm.at[wid * 5 + 4])


_sc_nms = functools.partial(
    pl.kernel,
    out_type=jax.ShapeDtypeStruct((NFG * 5, NP), jnp.float32),
    mesh=_mesh,
    scratch_types=[
        pltpu.VMEM((NP,), jnp.float32),
        pltpu.VMEM((NP,), jnp.float32),
        pltpu.VMEM((NP,), jnp.float32),
        pltpu.VMEM((NP,), jnp.float32),
        pltpu.VMEM((NP,), jnp.float32),
        pltpu.VMEM((NP,), jnp.float32),
        pltpu.VMEM((NP,), jnp.float32),
        pltpu.VMEM((NP,), jnp.float32),
        pltpu.VMEM((NP,), jnp.float32),
        pltpu.VMEM((NP,), jnp.float32),
        pltpu.VMEM((NP,), jnp.int32),
        pltpu.VMEM((NP,), jnp.float32),
        pltpu.VMEM((320,), jnp.float32),
        pltpu.VMEM((NP,), jnp.float32),
        pltpu.VMEM((NP,), jnp.float32),
        pltpu.VMEM((NP,), jnp.float32),
        pltpu.VMEM((NP,), jnp.float32),
        pltpu.VMEM((NP,), jnp.float32),
        pltpu.SemaphoreType.DMA,
    ],
    compiler_params=pltpu.CompilerParams(needs_layout_passes=False),
)(_nms_body)


def kernel(raw_cls_bbox, raw_prob):
    cls_bbox = raw_cls_bbox.reshape(N, NCLS, 4)
    boxes_t = jnp.transpose(cls_bbox[:, 1:, :], (1, 2, 0))
    boxes_p = jnp.concatenate(
        [boxes_t, jnp.zeros((NFG, 4, NP - N), jnp.float32)],
        axis=-1).reshape(NFG * 4, NP)
    scores_t = jnp.transpose(raw_prob[:, 1:], (1, 0))
    scores_p = jnp.concatenate(
        [scores_t, jnp.full((NFG, NP - N), -1.0, jnp.float32)], axis=-1)
    out = _sc_nms(scores_p, boxes_p).reshape(NFG, 5, NP)
    return jnp.transpose(out[:, :, :N], (0, 2, 1))


# speedup vs baseline: 499.3140x; 1.0084x over previous
import functools

import jax
import jax.numpy as jnp
from jax import lax
from jax.experimental import pallas as pl
from jax.experimental.pallas import tpu as pltpu
from jax.experimental.pallas import tpu_sc as plsc

N = 5000
NCLS = 21
NFG = NCLS - 1
PROB_THRESH = 0.05
NMS_THRESH = 0.3
L = 16
NP = 5024
NCHUNKS = NP // L
NUM_CORES = 2
BIG_I32 = 2**30
T = 6
R = 8
TFRAC = NMS_THRESH / (1.0 + NMS_THRESH)

_mesh = plsc.VectorSubcoreMesh(core_axis_name="c", subcore_axis_name="s")


def _nms_body(scores_hbm, boxes_hbm, out_hbm,
              s_v, y1_v, x1_v, y2_v, x2_v,
              w_v, cy1_v, cx1_v, cy2_v, cx2_v, cidx_v, ca_v, cm_v,
              o0_v, o1_v, o2_v, o3_v, o4_v, sem):
    wid = lax.axis_index("s") * NUM_CORES + lax.axis_index("c")

    @pl.when(wid < NFG)
    def _run():
        lane_iota = lax.iota(jnp.int32, L)
        lane0 = lane_iota == 0
        neg1 = jnp.full((L,), -1.0, jnp.float32)
        zero16 = jnp.zeros((L,), jnp.float32)

        cps = [
            pltpu.async_copy(scores_hbm.at[wid], s_v, sem),
            pltpu.async_copy(boxes_hbm.at[wid * 4 + 0], y1_v, sem),
            pltpu.async_copy(boxes_hbm.at[wid * 4 + 1], x1_v, sem),
            pltpu.async_copy(boxes_hbm.at[wid * 4 + 2], y2_v, sem),
            pltpu.async_copy(boxes_hbm.at[wid * 4 + 3], x2_v, sem),
        ]

        @plsc.parallel_loop(0, NCHUNKS, 1, unroll=4)
        def init_body(i):
            d = pl.ds(pl.multiple_of(i * L, L), L)
            w_v[d] = neg1
            o0_v[d] = zero16
            o1_v[d] = zero16
            o2_v[d] = zero16
            o3_v[d] = zero16
            o4_v[d] = zero16

        def cm_init(i, carry):
            cm_v[pl.ds(pl.multiple_of(i * L, L), L)] = neg1
            return carry
        lax.fori_loop(0, NCHUNKS // L + 1, cm_init, 0)

        for cp in cps:
            cp.wait()

        def compact_body(i, base):
            d = pl.ds(pl.multiple_of(i * L, L), L)
            sv = s_v[d]
            msk = sv > PROB_THRESH
            pos = plsc.cumsum(msk.astype(jnp.int32))
            cnt = jnp.max(pos)
            dest = pos + (base - 1)
            v1 = y1_v[d]
            v2 = x1_v[d]
            v3 = y2_v[d]
            v4 = x2_v[d]
            plsc.store_scatter(w_v, [dest], sv, mask=msk)
            plsc.store_scatter(cy1_v, [dest], v1, mask=msk)
            plsc.store_scatter(cx1_v, [dest], v2, mask=msk)
            plsc.store_scatter(cy2_v, [dest], v3, mask=msk)
            plsc.store_scatter(cx2_v, [dest], v4, mask=msk)
            plsc.store_scatter(cidx_v, [dest], lane_iota + i * L, mask=msk)
            plsc.store_scatter(ca_v, [dest],
                               TFRAC * ((v3 - v1) * (v4 - v2)), mask=msk)
            return base + cnt
        m_count = lax.fori_loop(0, NCHUNKS, compact_body, 0)
        nch0 = (m_count + (L - 1)) // L

        @plsc.parallel_loop(0, nch0, 1, unroll=4)
        def cm_seed(i):
            wv = w_v[pl.ds(pl.multiple_of(i * L, L), L)]
            plsc.store_scatter(cm_v, [jnp.full((L,), i, jnp.int32)],
                               jnp.full((L,), jnp.max(wv), jnp.float32),
                               mask=lane0)

        def global_max(ncv):
            def b(i, acc):
                return jnp.maximum(acc, cm_v[pl.ds(pl.multiple_of(i * L, L), L)])
            return jnp.max(lax.fori_loop(0, ncv, b, neg1))

        def max_locate(ncv):
            def fb(i, acc):
                acc_v, acc_p = acc
                v = cm_v[pl.ds(pl.multiple_of(i * L, L), L)]
                gt = v > acc_v
                return (jnp.maximum(acc_v, v),
                        jnp.where(gt, lane_iota + i * L, acc_p))
            acc_v, acc_p = lax.fori_loop(
                0, ncv, fb,
                (jnp.full((L,), -2.0, jnp.float32),
                 jnp.full((L,), BIG_I32, jnp.int32)))
            m_k = jnp.max(acc_v)
            ci = jnp.min(jnp.where(acc_v == m_k, acc_p, BIG_I32))
            wv = w_v[pl.ds(pl.multiple_of(ci * L, L), L)]
            lane = jnp.min(jnp.where(wv == m_k, lane_iota, BIG_I32))
            return m_k, ci, wv, lane

        def round_cond(carry):
            return carry[0] > PROB_THRESH

        def round_body(carry):
            m0, nch_old = carry

            def rc_body(i, base):
                d = pl.ds(pl.multiple_of(i * L, L), L)
                wv = w_v[d]
                v1 = cy1_v[d]
                v2 = cx1_v[d]
                v3 = cy2_v[d]
                v4 = cx2_v[d]
                vi = cidx_v[d]
                va = ca_v[d]
                msk = wv > PROB_THRESH
                pos = plsc.cumsum(msk.astype(jnp.int32))
                cnt = jnp.max(pos)
                dest = pos + (base - 1)
                plsc.store_scatter(w_v, [dest], wv, mask=msk)
                plsc.store_scatter(cy1_v, [dest], v1, mask=msk)
                plsc.store_scatter(cx1_v, [dest], v2, mask=msk)
                plsc.store_scatter(cy2_v, [dest], v3, mask=msk)
                plsc.store_scatter(cx2_v, [dest], v4, mask=msk)
                plsc.store_scatter(cidx_v, [dest], vi, mask=msk)
                plsc.store_scatter(ca_v, [dest], va, mask=msk)
                return base + cnt
            alive = lax.fori_loop(0, nch_old, rc_body, 0)
            nch = (alive + (L - 1)) // L

            @plsc.parallel_loop(0, nch_old, 1, unroll=4)
            def cl_body(i):
                d = pl.ds(pl.multiple_of(i * L, L), L)
                wv = jnp.where(lane_iota + i * L < alive, w_v[d], -1.0)
                w_v[d] = wv
                plsc.store_scatter(cm_v, [jnp.full((L,), i, jnp.int32)],
                                   jnp.full((L,), jnp.max(wv), jnp.float32),
                                   mask=lane0)

            ncv = (nch + (L - 1)) // L

            def iter_body(_, m):
                @pl.when(m > PROB_THRESH)
                def _it():
                    lead = []
                    for k in range(T):
                        m_k, ci, wv, lane = max_locate(ncv)
                        j = ci * L + lane
                        jv = jnp.full((L,), j, jnp.int32)
                        ly1 = plsc.load_gather(cy1_v, [jv])
                        lx1 = plsc.load_gather(cx1_v, [jv])
                        ly2 = plsc.load_gather(cy2_v, [jv])
                        lx2 = plsc.load_gather(cx2_v, [jv])
                        lidx = plsc.load_gather(cidx_v, [jv])
                        nw = jnp.where(lane_iota == lane, -1.0, wv)
                        w_v[pl.ds(pl.multiple_of(ci * L, L), L)] = nw
                        plsc.store_scatter(
                            cm_v, [jnp.full((L,), ci, jnp.int32)],
                            jnp.full((L,), jnp.max(nw), jnp.float32),
                            mask=lane0)
                        valid = jnp.full((L,), m_k > PROB_THRESH)
                        lead.append((ly1, lx1, ly2, lx2, lidx, m_k, valid))

                    def ovl(a, b):
                        ay1, ax1, ay2, ax2 = a[0], a[1], a[2], a[3]
                        by1, bx1, by2, bx2 = b[0], b[1], b[2], b[3]
                        ih = jnp.maximum(
                            jnp.minimum(ay2, by2) - jnp.maximum(ay1, by1), 0.0)
                        iw = jnp.maximum(
                            jnp.minimum(ax2, bx2) - jnp.maximum(ax1, bx1), 0.0)
                        inter = ih * iw
                        aa = (ay2 - ay1) * (ax2 - ax1)
                        ab = (by2 - by1) * (bx2 - bx1)
                        return inter > NMS_THRESH * (aa + ab - inter)

                    acc = []
                    for k in range(T):
                        a_k = lead[k][6]
                        for kk in range(k):
                            a_k = a_k & ~(acc[kk] & ovl(lead[kk], lead[k]))
                        acc.append(a_k)

                    eff = []
                    for k in range(T):
                        ly1, lx1, ly2, lx2, lidx, m_k, _ = lead[k]
                        mk = lane0 & acc[k]
                        plsc.store_scatter(o0_v, [lidx], ly1, mask=mk)
                        plsc.store_scatter(o1_v, [lidx], lx1, mask=mk)
                        plsc.store_scatter(o2_v, [lidx], ly2, mask=mk)
                        plsc.store_scatter(o3_v, [lidx], lx2, mask=mk)
                        plsc.store_scatter(o4_v, [lidx],
                                           jnp.full((L,), m_k, jnp.float32),
                                           mask=mk)
                        eff.append((jnp.where(acc[k], ly1, 0.0),
                                    jnp.where(acc[k], lx1, 0.0),
                                    jnp.where(acc[k], ly2, 0.0),
                                    jnp.where(acc[k], lx2, 0.0)))

                    lca = [TFRAC * (e[2] - e[0]) * (e[3] - e[1])
                           for e in eff]

                    @plsc.parallel_loop(0, nch, 1, unroll=4)
                    def sb(i):
                        d = pl.ds(pl.multiple_of(i * L, L), L)
                        w = w_v[d]
                        y1 = cy1_v[d]
                        x1 = cx1_v[d]
                        y2 = cy2_v[d]
                        x2 = cx2_v[d]
                        pa = ca_v[d]
                        sup = None
                        for k in range(T):
                            ey1, ex1, ey2, ex2 = eff[k]
                            ih = jnp.maximum(jnp.minimum(y2, ey2)
                                             - jnp.maximum(y1, ey1), 0.0)
                            iw = jnp.maximum(jnp.minimum(x2, ex2)
                                             - jnp.maximum(x1, ex1), 0.0)
                            inter = ih * iw
                            s_k = inter > pa + lca[k]
                            sup = s_k if sup is None else (sup | s_k)
                        nw = jnp.where(sup, -1.0, w)
                        w_v[d] = nw
                        plsc.store_scatter(
                            cm_v, [jnp.full((L,), i, jnp.int32)],
                            jnp.full((L,), jnp.max(nw), jnp.float32),
                            mask=lane0)
                return global_max(ncv)

            m_out = lax.fori_loop(0, R, iter_body, m0)
            return (m_out, nch)

        lax.while_loop(round_cond, round_body, (global_max((nch0 + L - 1) // L),
                                                nch0))

        pltpu.sync_copy(o0_v, out_hbm.at[wid * 5 + 0])
        pltpu.sync_copy(o1_v, out_hbm.at[wid * 5 + 1])
        pltpu.sync_copy(o2_v, out_hbm.at[wid * 5 + 2])
        pltpu.sync_copy(o3_v, out_hbm.at[wid * 5 + 3])
        pltpu.sync_copy(o4_v, out_hbm.at[wid * 5 + 4])


_sc_nms = functools.partial(
    pl.kernel,
    out_type=jax.ShapeDtypeStruct((NFG * 5, NP), jnp.float32),
    mesh=_mesh,
    scratch_types=[
        pltpu.VMEM((NP,), jnp.float32),
        pltpu.VMEM((NP,), jnp.float32),
        pltpu.VMEM((NP,), jnp.float32),
        pltpu.VMEM((NP,), jnp.float32),
        pltpu.VMEM((NP,), jnp.float32),
        pltpu.VMEM((NP,), jnp.float32),
        pltpu.VMEM((NP,), jnp.float32),
        pltpu.VMEM((NP,), jnp.float32),
        pltpu.VMEM((NP,), jnp.float32),
        pltpu.VMEM((NP,), jnp.float32),
        pltpu.VMEM((NP,), jnp.int32),
        pltpu.VMEM((NP,), jnp.float32),
        pltpu.VMEM((320,), jnp.float32),
        pltpu.VMEM((NP,), jnp.float32),
        pltpu.VMEM((NP,), jnp.float32),
        pltpu.VMEM((NP,), jnp.float32),
        pltpu.VMEM((NP,), jnp.float32),
        pltpu.VMEM((NP,), jnp.float32),
        pltpu.SemaphoreType.DMA,
    ],
    compiler_params=pltpu.CompilerParams(needs_layout_passes=False),
)(_nms_body)


def kernel(raw_cls_bbox, raw_prob):
    cls_bbox = raw_cls_bbox.reshape(N, NCLS, 4)
    boxes_t = jnp.transpose(cls_bbox[:, 1:, :], (1, 2, 0))
    boxes_p = jnp.concatenate(
        [boxes_t, jnp.zeros((NFG, 4, NP - N), jnp.float32)],
        axis=-1).reshape(NFG * 4, NP)
    scores_t = jnp.transpose(raw_prob[:, 1:], (1, 0))
    scores_p = jnp.concatenate(
        [scores_t, jnp.full((NFG, NP - N), -1.0, jnp.float32)], axis=-1)
    out = _sc_nms(scores_p, boxes_p).reshape(NFG, 5, NP)
    return jnp.transpose(out[:, :, :N], (0, 2, 1))
